# Initial kernel scaffold; baseline (speedup 1.0000x reference)
import jax
import jax.numpy as jnp
from jax.experimental import pallas as pl


def kernel(x, w_gating):
    raise NotImplementedError("write your pallas kernel here")


# speedup vs baseline: 5.6890x; 5.6890x over previous
import functools

import jax
import jax.numpy as jnp
from jax.experimental import pallas as pl
from jax.experimental.pallas import tpu as pltpu

_MAX_K = 8
_TAU = 0.7
_BLOCK_T = 512


def _gating_body(x_ref, w_ref, out_ref):
    xb = x_ref[...]
    w = w_ref[...]
    logits = jax.lax.dot_general(
        xb, w, (((1,), (0,)), ((), ())), preferred_element_type=jnp.float32)
    b, e = logits.shape
    mx = jnp.max(logits, axis=-1, keepdims=True)
    ex = jnp.exp(logits - mx)
    p = ex / jnp.sum(ex, axis=-1, keepdims=True)

    iota = jax.lax.broadcasted_iota(jnp.int32, (b, e), 1)
    q = p
    cum = jnp.zeros((b, 1), jnp.float32)
    denom = jnp.zeros((b, 1), jnp.float32)
    combine = jnp.zeros((b, e), jnp.float32)
    for _ in range(_MAX_K):
        mk = jnp.max(q, axis=-1, keepdims=True)
        idx = jnp.min(jnp.where(q == mk, iota, e), axis=-1, keepdims=True)
        take = iota == idx
        cum_new = cum + mk
        keep = (cum_new - mk) < _TAU
        gate = jnp.where(keep, mk, 0.0)
        combine = combine + jnp.where(take, gate, 0.0)
        denom = denom + gate
        q = jnp.where(take, -1.0, q)
        cum = cum_new
    out_ref[...] = combine / (denom + 1e-9)


@jax.jit
def kernel(x, w_gating):
    t, d = x.shape
    e = w_gating.shape[1]
    grid = t // _BLOCK_T
    return pl.pallas_call(
        _gating_body,
        grid=(grid,),
        in_specs=[
            pl.BlockSpec((_BLOCK_T, d), lambda i: (i, 0)),
            pl.BlockSpec((d, e), lambda i: (0, 0)),
        ],
        out_specs=pl.BlockSpec((_BLOCK_T, e), lambda i: (i, 0)),
        out_shape=jax.ShapeDtypeStruct((t, e), jnp.float32),
    )(x, w_gating)
